# baseline (speedup 1.0000x reference)
import functools

import jax
import jax.numpy as jnp
from jax import lax
from jax.experimental import pallas as pl
from jax.experimental.pallas import tpu as pltpu
from jax.experimental.pallas import tpu_sc as plsc

N = 10000
E = 160000
D = 128
DE = 16
H = 256
HE = 128
L = 4
V = 1000
B = 256

NC = 2
NS = 16
NHALF = N // NC
TRASH = NHALF
YROWS = NHALF + 8
W = 128
CH = 128
NCHUNK = E // CH
CPT = 80
NCHP = CPT * NS
RPT = 320
NB = 3


def _prep_body(ei_ref, srcadj_ref, linidx_ref, dstadj_ref):
    src = ei_ref[0]
    dst = ei_ref[1]
    pad_chunks = NCHP - NCHUNK
    zpad = jnp.zeros((pad_chunks, CH), jnp.int32)
    for e in range(4):
        for j in range(2):
            srcadj_ref[2 * e + j] = jnp.concatenate(
                [2 * src + (2 * e * N + j), zpad], axis=0)
    eid = (lax.broadcasted_iota(jnp.int32, (NCHUNK, CH), 0) * CH
           + lax.broadcasted_iota(jnp.int32, (NCHUNK, CH), 1))
    for e in range(4):
        linidx_ref[e] = jnp.concatenate([eid + e * E, zpad], axis=0)
    tpad = jnp.full((pad_chunks, CH), TRASH, jnp.int32)
    for c in range(NC):
        d2 = dst - c * NHALF
        d2 = jnp.where((d2 >= 0) & (d2 < NHALF), d2, TRASH)
        dstadj_ref[c] = jnp.concatenate([d2, tpad], axis=0)


def _prep(edge_index):
    ei = edge_index.reshape(2, NCHUNK, CH)
    return pl.pallas_call(
        _prep_body,
        grid=(1,),
        in_specs=[pl.BlockSpec((2, NCHUNK, CH), lambda i: (0, 0, 0))],
        out_specs=[pl.BlockSpec((8, NCHP, CH), lambda i: (0, 0, 0)),
                   pl.BlockSpec((4, NCHP, CH), lambda i: (0, 0, 0)),
                   pl.BlockSpec((NC, NCHP, CH), lambda i: (0, 0, 0))],
        out_shape=[jax.ShapeDtypeStruct((8, NCHP, CH), jnp.int32),
                   jax.ShapeDtypeStruct((4, NCHP, CH), jnp.int32),
                   jax.ShapeDtypeStruct((NC, NCHP, CH), jnp.int32)],
    )(ei)


def _make_sc_segsum(npass):
    def body(table_hbm, idx_hbm, dstadj_hbm, out_hbm,
             y_sh, src_slab, dst_slab, rows_v, zero_v, sems):
        cid = lax.axis_index("c")
        sid = lax.axis_index("s")
        zv = jnp.zeros((16,), jnp.float32)
        for i in range(8):
            for j in range(W // 16):
                zero_v[i, pl.ds(j * 16, 16)] = zv
        pltpu.sync_copy(dstadj_hbm.at[cid].at[pl.ds(sid * CPT, CPT)],
                        dst_slab)

        for p in range(npass):
            @pl.when(sid < NS - 1)
            def _():
                for k in range(RPT // 8):
                    pltpu.sync_copy(
                        zero_v, y_sh.at[pl.ds(sid * RPT + k * 8, 8)])

            @pl.when(sid == NS - 1)
            def _():
                for k in range((YROWS - (NS - 1) * RPT) // 8):
                    pltpu.sync_copy(
                        zero_v, y_sh.at[pl.ds((NS - 1) * RPT + k * 8, 8)])
            plsc.subcore_barrier()
            pltpu.sync_copy(idx_hbm.at[p].at[pl.ds(sid * CPT, CPT)],
                            src_slab)
            for b in range(NB):
                pltpu.async_copy(table_hbm.at[src_slab.at[b]], rows_v.at[b],
                                 sems.at[b])

            def chunk_step(k, _):
                b = lax.rem(k, NB)
                pltpu.make_async_copy(table_hbm.at[src_slab.at[0]],
                                      rows_v.at[b], sems.at[b]).wait()
                pltpu.sync_copy(rows_v.at[b], y_sh.at[dst_slab.at[k]],
                                add=True)

                @pl.when(k + NB < CPT)
                def _():
                    pltpu.async_copy(table_hbm.at[src_slab.at[k + NB]],
                                     rows_v.at[b], sems.at[b])
                return 0

            lax.fori_loop(0, CPT, chunk_step, 0, unroll=False)
            plsc.subcore_barrier()
            @pl.when(sid < NS - 1)
            def _():
                pltpu.sync_copy(
                    y_sh.at[pl.ds(sid * RPT, RPT)],
                    out_hbm.at[p].at[pl.ds(cid * NHALF + sid * RPT, RPT)])

            @pl.when(sid == NS - 1)
            def _():
                last = NHALF - (NS - 1) * RPT
                pltpu.sync_copy(
                    y_sh.at[pl.ds((NS - 1) * RPT, last)],
                    out_hbm.at[p].at[pl.ds(cid * NHALF + (NS - 1) * RPT,
                                           last)])
            plsc.subcore_barrier()

    return pl.kernel(
        body,
        out_type=jax.ShapeDtypeStruct((npass, N, W), jnp.float32),
        mesh=plsc.VectorSubcoreMesh(core_axis_name="c", subcore_axis_name="s",
                                    num_cores=NC, num_subcores=NS),
        scratch_types=[
            pltpu.VMEM_SHARED((YROWS, W), jnp.float32),
            pltpu.VMEM((CPT, CH), jnp.int32),
            pltpu.VMEM((CPT, CH), jnp.int32),
            pltpu.VMEM((NB, CH, W), jnp.float32),
            pltpu.VMEM((8, W), jnp.float32),
            pltpu.SemaphoreType.DMA((NB,)),
        ],
    )


_spmm8 = _make_sc_segsum(8)
_hesum4 = _make_sc_segsum(4)


def _mm_body(a_ref, b_ref, o_ref):
    o_ref[...] = jnp.dot(a_ref[...], b_ref[...],
                         preferred_element_type=jnp.float32)


def _matmul(a, b, bm=512):
    M, K = a.shape
    K2, Nc = b.shape
    assert K == K2
    Mp = ((M + bm - 1) // bm) * bm
    if Mp != M:
        a = jnp.pad(a, ((0, Mp - M), (0, 0)))
    out = pl.pallas_call(
        _mm_body,
        grid=(Mp // bm,),
        in_specs=[pl.BlockSpec((bm, K), lambda i: (i, 0)),
                  pl.BlockSpec((K, Nc), lambda i: (0, 0))],
        out_specs=pl.BlockSpec((bm, Nc), lambda i: (i, 0)),
        out_shape=jax.ShapeDtypeStruct((Mp, Nc), jnp.float32),
    )(a, b)
    return out[:M] if Mp != M else out


def _mlp(p, x):
    return jnp.maximum(_matmul(x, p['W1']) + p['b1'], 0.0) @ p['W2'] + p['b2']


def _encoders(params, x_node, x_edge, src, dst, srcadj, dstadj):
    encs = [params['enc_act'], params['enc_del'],
            params['enc_add'], params['enc_arm']]
    hs = []
    agg_ehs = []
    for p in encs:
        h = jnp.maximum(_matmul(x_node, p['W_in']) + p['b_in'], 0.0)
        he = jnp.maximum(_matmul(x_edge, p['W_e_in']) + p['b_e_in'], 0.0)
        agg_ehs.append(_matmul(jax.ops.segment_sum(he, dst, num_segments=N),
                               p['W_eh']))
        hs.append(h)
    for l in range(L):
        hm = jnp.stack([_matmul(hs[i], encs[i]['W_msg'][l]) for i in range(4)])
        y = _spmm8(hm.reshape(8 * N, W), srcadj, dstadj)
        hs = [jnp.maximum(_matmul(hs[i], encs[i]['W_self'][l])
                          + jnp.concatenate([y[2 * i], y[2 * i + 1]], axis=-1)
                          + agg_ehs[i] + encs[i]['b'][l], 0.0)
              for i in range(4)]
    return hs


def _set2set(p, h_node, batch):
    q_star = jnp.zeros((B, 2 * H), dtype=h_node.dtype)
    h0 = jnp.zeros((B, H), dtype=h_node.dtype)
    c0 = h0
    h1 = h0
    c1 = h0
    for _ in range(6):
        g = q_star @ p['W_ih0'].T + h0 @ p['W_hh0'].T + p['b0']
        i, f, gg, o = jnp.split(g, 4, axis=-1)
        c0 = jax.nn.sigmoid(f) * c0 + jax.nn.sigmoid(i) * jnp.tanh(gg)
        h0 = jax.nn.sigmoid(o) * jnp.tanh(c0)
        g = h0 @ p['W_ih1'].T + h1 @ p['W_hh1'].T + p['b1']
        i, f, gg, o = jnp.split(g, 4, axis=-1)
        c1 = jax.nn.sigmoid(f) * c1 + jax.nn.sigmoid(i) * jnp.tanh(gg)
        h1 = jax.nn.sigmoid(o) * jnp.tanh(c1)
        q = h1
        e = jnp.sum(h_node * q[batch], axis=-1)
        emax = jax.ops.segment_max(e, batch, num_segments=B)
        emax = jnp.where(jnp.isfinite(emax), emax, 0.0)
        ex = jnp.exp(e - emax[batch])
        denom = jax.ops.segment_sum(ex, batch, num_segments=B)
        a = ex / (denom[batch] + 1e-12)
        r = jax.ops.segment_sum(a[:, None] * h_node, batch, num_segments=B)
        q_star = jnp.concatenate([q, r], axis=-1)
    return q_star


def kernel(x_node, x_edge, edge_index, node2graph, params):
    src = edge_index[0]
    dst = edge_index[1]
    srcadj, linidx, dstadj = _prep(edge_index)
    h_act, h_del, h_add, h_arm = _encoders(
        params, x_node, x_edge, src, dst, srcadj, dstadj)
    q = _set2set(params['s2s'], h_act, node2graph)
    pred_act = _mlp(params['cls_act'], q)
    h_edge = _mlp(params['edge_mlp'], x_edge)
    pred_del = _mlp(params['cls_del'],
                    jnp.concatenate([h_del[src], h_edge, h_del[dst]], axis=1))
    pred_add = _mlp(params['cls_add'], h_add)
    pred_arm = _mlp(params['cls_arm'], h_arm)
    return (pred_act, pred_del, pred_add, pred_arm)


# speedup vs baseline: 1.3717x; 1.3717x over previous
import functools

import jax
import jax.numpy as jnp
from jax import lax
from jax.experimental import pallas as pl
from jax.experimental.pallas import tpu as pltpu
from jax.experimental.pallas import tpu_sc as plsc

N = 10000
E = 160000
D = 128
DE = 16
H = 256
HE = 128
L = 4
V = 1000
B = 256

NC = 2
NS = 16
TRASH = N
YROWS = 10024
W = 128
CH = 128
NCHUNK = E // CH
CPT = 40
NCHP = CPT * NS * NC
RPT = 632
NB = 2


def _prep_body(ei_ref, srcadj_ref, linidx_ref, dstadj_ref):
    src = ei_ref[0]
    dst = ei_ref[1]
    pad_chunks = NCHP - NCHUNK
    zpad = jnp.zeros((pad_chunks, CH), jnp.int32)
    for e in range(4):
        for j in range(2):
            srcadj_ref[2 * e + j] = jnp.concatenate(
                [2 * src + (2 * e * N + j), zpad], axis=0)
    eid = (lax.broadcasted_iota(jnp.int32, (NCHUNK, CH), 0) * CH
           + lax.broadcasted_iota(jnp.int32, (NCHUNK, CH), 1))
    for e in range(4):
        linidx_ref[e] = jnp.concatenate([eid + e * E, zpad], axis=0)
    tpad = jnp.full((pad_chunks, CH), TRASH, jnp.int32)
    dstadj_ref[...] = jnp.concatenate([dst, tpad], axis=0)


def _prep(edge_index):
    ei = edge_index.reshape(2, NCHUNK, CH)
    return pl.pallas_call(
        _prep_body,
        grid=(1,),
        in_specs=[pl.BlockSpec((2, NCHUNK, CH), lambda i: (0, 0, 0))],
        out_specs=[pl.BlockSpec((8, NCHP, CH), lambda i: (0, 0, 0)),
                   pl.BlockSpec((4, NCHP, CH), lambda i: (0, 0, 0)),
                   pl.BlockSpec((NCHP, CH), lambda i: (0, 0))],
        out_shape=[jax.ShapeDtypeStruct((8, NCHP, CH), jnp.int32),
                   jax.ShapeDtypeStruct((4, NCHP, CH), jnp.int32),
                   jax.ShapeDtypeStruct((NCHP, CH), jnp.int32)],
    )(ei)


def _make_sc_segsum(npass):
    def body(table_hbm, idx_hbm, dstadj_hbm, out_hbm,
             y_sh, src_slab, dst_slab, rows_v, zero_v, gsem, ssem, zsem):
        cid = lax.axis_index("c")
        sid = lax.axis_index("s")
        crow = cid * (NS * CPT) + sid * CPT
        zv = jnp.zeros((16,), jnp.float32)
        for i in range(32):
            for j in range(W // 16):
                zero_v[i, pl.ds(j * 16, 16)] = zv
        pltpu.sync_copy(dstadj_hbm.at[pl.ds(crow, CPT)], dst_slab)

        last_rows = YROWS - (NS - 1) * RPT

        for p in range(npass):
            @pl.when(sid < NS - 1)
            def _():
                for k in range(RPT // 32):
                    pltpu.async_copy(
                        zero_v, y_sh.at[pl.ds(sid * RPT + k * 32, 32)], zsem)
                for k in range(RPT // 32):
                    pltpu.make_async_copy(
                        zero_v, y_sh.at[pl.ds(sid * RPT + k * 32, 32)],
                        zsem).wait()

            @pl.when(sid == NS - 1)
            def _():
                for k in range(last_rows // 32):
                    pltpu.async_copy(
                        zero_v, y_sh.at[pl.ds((NS - 1) * RPT + k * 32, 32)],
                        zsem)
                for k in range(last_rows // 32):
                    pltpu.make_async_copy(
                        zero_v, y_sh.at[pl.ds((NS - 1) * RPT + k * 32, 32)],
                        zsem).wait()
            plsc.subcore_barrier()
            pltpu.sync_copy(idx_hbm.at[p].at[pl.ds(crow, CPT)], src_slab)
            for b in range(NB):
                pltpu.async_copy(table_hbm.at[src_slab.at[b]], rows_v.at[b],
                                 gsem.at[b])

            def chunk_step(k, _):
                b = lax.rem(k, NB)
                pltpu.make_async_copy(table_hbm.at[src_slab.at[0]],
                                      rows_v.at[b], gsem.at[b]).wait()
                pltpu.async_copy(rows_v.at[b], y_sh.at[dst_slab.at[k]],
                                 ssem.at[b], add=True)

                @pl.when(k >= 1)
                def _():
                    bp = lax.rem(k - 1, NB)
                    pltpu.make_async_copy(rows_v.at[bp],
                                          y_sh.at[dst_slab.at[0]],
                                          ssem.at[bp]).wait()

                    @pl.when(k - 1 + NB < CPT)
                    def _():
                        pltpu.async_copy(table_hbm.at[src_slab.at[k - 1 + NB]],
                                         rows_v.at[bp], gsem.at[bp])
                return 0

            lax.fori_loop(0, CPT, chunk_step, 0, unroll=False)
            bl = (CPT - 1) % NB
            pltpu.make_async_copy(rows_v.at[bl], y_sh.at[dst_slab.at[0]],
                                  ssem.at[bl]).wait()
            plsc.subcore_barrier()
            @pl.when(sid < NS - 1)
            def _():
                pltpu.sync_copy(
                    y_sh.at[pl.ds(sid * RPT, RPT)],
                    out_hbm.at[2 * p + cid].at[pl.ds(sid * RPT, RPT)])

            @pl.when(sid == NS - 1)
            def _():
                last = N - (NS - 1) * RPT
                pltpu.sync_copy(
                    y_sh.at[pl.ds((NS - 1) * RPT, last)],
                    out_hbm.at[2 * p + cid].at[pl.ds((NS - 1) * RPT, last)])
            plsc.subcore_barrier()

    return pl.kernel(
        body,
        out_type=jax.ShapeDtypeStruct((2 * npass, N, W), jnp.float32),
        mesh=plsc.VectorSubcoreMesh(core_axis_name="c", subcore_axis_name="s",
                                    num_cores=NC, num_subcores=NS),
        scratch_types=[
            pltpu.VMEM_SHARED((YROWS, W), jnp.float32),
            pltpu.VMEM((CPT, CH), jnp.int32),
            pltpu.VMEM((CPT, CH), jnp.int32),
            pltpu.VMEM((NB, CH, W), jnp.float32),
            pltpu.VMEM((32, W), jnp.float32),
            pltpu.SemaphoreType.DMA((NB,)),
            pltpu.SemaphoreType.DMA((NB,)),
            pltpu.SemaphoreType.DMA,
        ],
    )


_spmm8 = _make_sc_segsum(8)
_hesum4 = _make_sc_segsum(4)


def _mm_body(a_ref, b_ref, o_ref):
    o_ref[...] = jnp.dot(a_ref[...], b_ref[...],
                         preferred_element_type=jnp.float32)


def _matmul(a, b, bm=512):
    M, K = a.shape
    K2, Nc = b.shape
    assert K == K2
    Mp = ((M + bm - 1) // bm) * bm
    if Mp != M:
        a = jnp.pad(a, ((0, Mp - M), (0, 0)))
    out = pl.pallas_call(
        _mm_body,
        grid=(Mp // bm,),
        in_specs=[pl.BlockSpec((bm, K), lambda i: (i, 0)),
                  pl.BlockSpec((K, Nc), lambda i: (0, 0))],
        out_specs=pl.BlockSpec((bm, Nc), lambda i: (i, 0)),
        out_shape=jax.ShapeDtypeStruct((Mp, Nc), jnp.float32),
    )(a, b)
    return out[:M] if Mp != M else out


def _mlp(p, x):
    return jnp.maximum(_matmul(x, p['W1']) + p['b1'], 0.0) @ p['W2'] + p['b2']


def _encoders(params, x_node, x_edge, src, dst, srcadj, linidx, dstadj):
    encs = [params['enc_act'], params['enc_del'],
            params['enc_add'], params['enc_arm']]
    hs = []
    hes = []
    for p in encs:
        h = jnp.maximum(_matmul(x_node, p['W_in']) + p['b_in'], 0.0)
        he = jnp.maximum(_matmul(x_edge, p['W_e_in']) + p['b_e_in'], 0.0)
        hes.append(he)
        hs.append(h)
    hesum = _hesum4(jnp.stack(hes).reshape(4 * E, HE), linidx, dstadj)
    agg_ehs = [_matmul(hesum[2 * i] + hesum[2 * i + 1], encs[i]['W_eh'])
               for i in range(4)]
    for l in range(L):
        hm = jnp.stack([_matmul(hs[i], encs[i]['W_msg'][l]) for i in range(4)])
        y = _spmm8(hm.reshape(8 * N, W), srcadj, dstadj)
        hs = [jnp.maximum(_matmul(hs[i], encs[i]['W_self'][l])
                          + jnp.concatenate([y[4 * i] + y[4 * i + 1],
                                             y[4 * i + 2] + y[4 * i + 3]],
                                            axis=-1)
                          + agg_ehs[i] + encs[i]['b'][l], 0.0)
              for i in range(4)]
    return hs


def _set2set(p, h_node, batch):
    q_star = jnp.zeros((B, 2 * H), dtype=h_node.dtype)
    h0 = jnp.zeros((B, H), dtype=h_node.dtype)
    c0 = h0
    h1 = h0
    c1 = h0
    for _ in range(6):
        g = q_star @ p['W_ih0'].T + h0 @ p['W_hh0'].T + p['b0']
        i, f, gg, o = jnp.split(g, 4, axis=-1)
        c0 = jax.nn.sigmoid(f) * c0 + jax.nn.sigmoid(i) * jnp.tanh(gg)
        h0 = jax.nn.sigmoid(o) * jnp.tanh(c0)
        g = h0 @ p['W_ih1'].T + h1 @ p['W_hh1'].T + p['b1']
        i, f, gg, o = jnp.split(g, 4, axis=-1)
        c1 = jax.nn.sigmoid(f) * c1 + jax.nn.sigmoid(i) * jnp.tanh(gg)
        h1 = jax.nn.sigmoid(o) * jnp.tanh(c1)
        q = h1
        e = jnp.sum(h_node * q[batch], axis=-1)
        emax = jax.ops.segment_max(e, batch, num_segments=B)
        emax = jnp.where(jnp.isfinite(emax), emax, 0.0)
        ex = jnp.exp(e - emax[batch])
        denom = jax.ops.segment_sum(ex, batch, num_segments=B)
        a = ex / (denom[batch] + 1e-12)
        r = jax.ops.segment_sum(a[:, None] * h_node, batch, num_segments=B)
        q_star = jnp.concatenate([q, r], axis=-1)
    return q_star


def kernel(x_node, x_edge, edge_index, node2graph, params):
    src = edge_index[0]
    dst = edge_index[1]
    srcadj, linidx, dstadj = _prep(edge_index)
    h_act, h_del, h_add, h_arm = _encoders(
        params, x_node, x_edge, src, dst, srcadj, linidx, dstadj)
    q = _set2set(params['s2s'], h_act, node2graph)
    pred_act = _mlp(params['cls_act'], q)
    h_edge = _mlp(params['edge_mlp'], x_edge)
    pred_del = _mlp(params['cls_del'],
                    jnp.concatenate([h_del[src], h_edge, h_del[dst]], axis=1))
    pred_add = _mlp(params['cls_add'], h_add)
    pred_arm = _mlp(params['cls_arm'], h_arm)
    return (pred_act, pred_del, pred_add, pred_arm)
